# baseline (speedup 1.0000x reference)
import numpy as np
import jax
import jax.numpy as jnp
from jax.experimental import pallas as pl
from jax.experimental.pallas import tpu as pltpu

_MOD = 16


def _interp_mat(l_in, l_out):
    pos = np.arange(l_out, dtype=np.float64) * (l_in - 1) / (l_out - 1)
    lo = np.floor(pos).astype(np.int64)
    hi = np.minimum(lo + 1, l_in - 1)
    w = pos - lo
    m = np.zeros((l_out, l_in), np.float64)
    m[np.arange(l_out), lo] += 1.0 - w
    m[np.arange(l_out), hi] += w
    return m.astype(np.float32)


_MI = _interp_mat(16, 32)

_PIECES = [
    ('w1t', 192, 22), ('wd1t', 192, 64), ('w2t', 384, 64), ('wd2t', 96, 128),
    ('wr1', 64, 22), ('wr2', 128, 64), ('wr3', 32, 128), ('wf', 24, 32),
    ('caw1', 16, 22), ('caw2', 24, 11),
    ('wq', 64, 32), ('wk', 64, 32), ('wv', 64, 32), ('wo', 64, 64),
    ('mi', 32, 16),
    ('bias', 24, 128),
]
_OFF = {}
_r = 0
for _nm, _nr, _nc in _PIECES:
    _OFF[_nm] = _r
    _r += _nr
_ROWS_PACK = _r

_BIAS_ORDER = ['b1', 'bn1_g', 'bn1_b', 'bn1_m', 'bn1_v', 'br1', 'bd1', 'br2',
               'b2', 'bn2_g', 'bn2_b', 'bn2_m', 'bn2_v', 'bd2', 'br3', 'bf',
               'bq', 'bk', 'bv', 'bo']
_BOFF = {nm: _OFF['bias'] + i for i, nm in enumerate(_BIAS_ORDER)}


def _mm_tt(a, b):
    return jax.lax.dot_general(a, b, (((1,), (1,)), ((), ())),
                               preferred_element_type=jnp.float32)


def _cdot(x3, w):
    return jax.lax.dot_general(x3, w, (((1,), (1,)), ((), ())),
                               preferred_element_type=jnp.float32)


def _rd(x):
    r = pltpu.roll(x, 1, axis=0)
    idx = jax.lax.broadcasted_iota(jnp.int32, (x.shape[0], 1), 0)
    return jnp.where((idx % _MOD) == 0, 0.0, r)


def _ru(x):
    r = pltpu.roll(x, x.shape[0] - 1, axis=0)
    idx = jax.lax.broadcasted_iota(jnp.int32, (x.shape[0], 1), 0)
    return jnp.where((idx % _MOD) == (_MOD - 1), 0.0, r)


def _lrd(x):
    r = pltpu.roll(x, 1, axis=2)
    idx = jax.lax.broadcasted_iota(jnp.int32, (1, 1, x.shape[2]), 2)
    return jnp.where((idx % _MOD) == 0, 0.0, r)


def _lru(x):
    r = pltpu.roll(x, x.shape[2] - 1, axis=2)
    idx = jax.lax.broadcasted_iota(jnp.int32, (1, 1, x.shape[2]), 2)
    return jnp.where((idx % _MOD) == (_MOD - 1), 0.0, r)


def _encode_all(x_ref, pack_ref, out_ref):
    X0 = x_ref[0]
    X1 = x_ref[1]
    X2 = x_ref[2]
    X3 = x_ref[3]

    mp = pack_ref[...]

    def piece(nm, r, c):
        o = _OFF[nm]
        return mp[o:o + r, 0:c]

    def brow(nm, c):
        o = _BOFF[nm]
        return mp[o:o + 1, 0:c]

    s1 = brow('bn1_g', 64) * jax.lax.rsqrt(brow('bn1_v', 64) + 1e-5)
    b1e = (brow('b1', 64) - brow('bn1_m', 64)) * s1 + brow('bn1_b', 64)
    s2 = brow('bn2_g', 128) * jax.lax.rsqrt(brow('bn2_v', 128) + 1e-5)
    b2e = (brow('b2', 128) - brow('bn2_m', 128)) * s2 + brow('bn2_b', 128)

    s1c = jnp.transpose(s1)
    w1t = piece('w1t', 192, 22)
    w10 = w1t[0:64] * s1c
    w11 = w1t[64:128] * s1c
    w12 = w1t[128:192] * s1c
    wr1 = piece('wr1', 64, 22)
    br1 = brow('br1', 64)

    def merge(t3):
        return t3.reshape(1024, t3.shape[2])

    h0 = merge(_cdot(_lrd(X3), w10) + _cdot(X0, w11) + _cdot(X1, w12)) + b1e
    h1 = merge(_cdot(X0, w10) + _cdot(X1, w11) + _cdot(X2, w12)) + b1e
    h2 = merge(_cdot(X1, w10) + _cdot(X2, w11) + _cdot(X3, w12)) + b1e
    h3 = merge(_cdot(X2, w10) + _cdot(X3, w11) + _cdot(_lru(X0), w12)) + b1e
    i0 = merge(_cdot(X0, wr1)) + br1
    i1 = merge(_cdot(X1, wr1)) + br1
    i2r = merge(_cdot(X2, wr1)) + br1
    i3r = merge(_cdot(X3, wr1)) + br1
    h0 = jnp.maximum(jnp.maximum(h0, 0.0) + i0, 0.0)
    h1 = jnp.maximum(jnp.maximum(h1, 0.0) + i1, 0.0)
    h2 = jnp.maximum(jnp.maximum(h2, 0.0) + i2r, 0.0)
    h3 = jnp.maximum(jnp.maximum(h3, 0.0) + i3r, 0.0)

    wd1t = piece('wd1t', 192, 64)
    v0 = wd1t[0:64]
    v1t = wd1t[64:128]
    v2t = wd1t[128:192]
    bd1r = brow('bd1', 64)
    ye = _mm_tt(_rd(h3), v0) + _mm_tt(h0, v1t) + _mm_tt(h1, v2t) + bd1r
    yo = _mm_tt(h1, v0) + _mm_tt(h2, v1t) + _mm_tt(h3, v2t) + bd1r

    wr2 = piece('wr2', 128, 64)
    br2 = brow('br2', 128)
    i2e = _mm_tt((i0 + i1) * 0.5, wr2) + br2
    i2o = _mm_tt((i2r + i3r) * 0.5, wr2) + br2

    s2c = jnp.transpose(s2)
    w2t = piece('w2t', 384, 64)
    c0 = w2t[0:128] * s2c
    c1 = w2t[128:256] * s2c
    c2 = w2t[256:384] * s2c
    he = _mm_tt(_rd(yo), c0) + _mm_tt(ye, c1) + _mm_tt(yo, c2) + b2e
    ho = _mm_tt(ye, c0) + _mm_tt(yo, c1) + _mm_tt(_ru(ye), c2) + b2e
    h4e = jnp.maximum(jnp.maximum(he, 0.0) + i2e, 0.0)
    h4o = jnp.maximum(jnp.maximum(ho, 0.0) + i2o, 0.0)

    wd2t = piece('wd2t', 96, 128)
    e0 = wd2t[0:32]
    e1 = wd2t[32:64]
    e2 = wd2t[64:96]
    z = _mm_tt(_rd(h4o), e0) + _mm_tt(h4e, e1) + _mm_tt(h4o, e2) \
        + brow('bd2', 32)
    i3 = _mm_tt((i2e + i2o) * 0.5, piece('wr3', 32, 128)) + brow('br3', 32)
    z2 = jnp.maximum(z + i3, 0.0)

    z3 = z2.reshape(64, 16, 32)
    hi = jax.lax.dot_general(z3, piece('mi', 32, 16), (((1,), (1,)), ((), ())),
                             preferred_element_type=jnp.float32)
    hf = jax.lax.dot_general(hi, piece('wf', 22, 32), (((1,), (1,)), ((), ())),
                             preferred_element_type=jnp.float32)
    hf = hf + brow('bf', 22)[None, :, :]

    avg = jnp.mean(hf, axis=1)
    mx = jnp.max(hf, axis=1)
    caw1 = piece('caw1', 11, 22)
    caw2 = piece('caw2', 22, 11)
    ga = _mm_tt(jnp.maximum(_mm_tt(avg, caw1), 0.0), caw2)
    gm = _mm_tt(jnp.maximum(_mm_tt(mx, caw1), 0.0), caw2)
    gate = jax.nn.sigmoid(ga + gm)
    o = jnp.sum(hf * gate[:, None, :], axis=2)

    o3 = o.reshape(8, 8, 32)
    q = jax.lax.dot_general(o3, piece('wq', 64, 32), (((2,), (1,)), ((), ())),
                            preferred_element_type=jnp.float32) \
        + brow('bq', 64)[None, :, :]
    k = jax.lax.dot_general(o3, piece('wk', 64, 32), (((2,), (1,)), ((), ())),
                            preferred_element_type=jnp.float32) \
        + brow('bk', 64)[None, :, :]
    v = jax.lax.dot_general(o3, piece('wv', 64, 32), (((2,), (1,)), ((), ())),
                            preferred_element_type=jnp.float32) \
        + brow('bv', 64)[None, :, :]
    sc = jax.lax.dot_general(q, k, (((2,), (2,)), ((0,), (0,))),
                             preferred_element_type=jnp.float32) * 0.125
    sc = sc - jnp.max(sc, axis=2, keepdims=True)
    es = jnp.exp(sc)
    p = es / jnp.sum(es, axis=2, keepdims=True)
    wvv = jax.lax.dot_general(p, v, (((2,), (1,)), ((0,), (0,))),
                              preferred_element_type=jnp.float32)
    pooled = jnp.mean(wvv, axis=1)
    out_ref[...] = _mm_tt(pooled, piece('wo', 64, 64)) + brow('bo', 64)


def kernel(x, params):
    p = params
    f32 = jnp.float32

    def rpad(t, rows):
        return jnp.pad(t, ((0, rows - t.shape[0]), (0, 128 - t.shape[1])))

    def kstack(w):
        return jnp.transpose(w, (2, 0, 1)).reshape(3 * w.shape[0], w.shape[1])

    pieces = [
        rpad(kstack(p['w1']), 192), rpad(kstack(p['wd1']), 192),
        rpad(kstack(p['w2']), 384), rpad(kstack(p['wd2']), 96),
        rpad(p['wr1'][:, :, 0], 64), rpad(p['wr2'][:, :, 0], 128),
        rpad(p['wr3'][:, :, 0], 32), rpad(p['wf'][:, :, 0], 24),
        rpad(p['ca_w1'], 16), rpad(p['ca_w2'], 24),
        rpad(p['wq'], 64), rpad(p['wk'], 64), rpad(p['wv'], 64),
        rpad(p['wo'], 64), rpad(jnp.asarray(_MI), 32),
    ]
    brows = [jnp.pad(p[nm], (0, 128 - p[nm].shape[0]))[None, :]
             for nm in _BIAS_ORDER]
    brows.append(jnp.zeros((4, 128), f32))
    pack = jnp.concatenate(pieces + brows, axis=0)

    xt = jnp.transpose(x.reshape(8, 22, 128, 4), (3, 0, 1, 2))

    out = pl.pallas_call(
        _encode_all,
        in_specs=[pl.BlockSpec((4, 8, 22, 128), lambda: (0, 0, 0, 0)),
                  pl.BlockSpec((_ROWS_PACK, 128), lambda: (0, 0))],
        out_specs=pl.BlockSpec((8, 64), lambda: (0, 0)),
        out_shape=jax.ShapeDtypeStruct((8, 64), f32),
    )(xt, pack)
    return out


# speedup vs baseline: 1.4952x; 1.4952x over previous
import numpy as np
import jax
import jax.numpy as jnp
from jax.experimental import pallas as pl
from jax.experimental.pallas import tpu as pltpu

_N = 1024
_MOD = 16


def _interp_mat(l_in, l_out):
    pos = np.arange(l_out, dtype=np.float64) * (l_in - 1) / (l_out - 1)
    lo = np.floor(pos).astype(np.int64)
    hi = np.minimum(lo + 1, l_in - 1)
    w = pos - lo
    m = np.zeros((l_out, l_in), np.float64)
    m[np.arange(l_out), lo] += 1.0 - w
    m[np.arange(l_out), hi] += w
    return m.astype(np.float32)


_MI = _interp_mat(16, 32)


def _mm(a, b):
    return jax.lax.dot_general(a, b, (((1,), (0,)), ((), ())),
                               preferred_element_type=jnp.float32)


def _mm_tt(a, b):
    return jax.lax.dot_general(a, b, (((1,), (1,)), ((), ())),
                               preferred_element_type=jnp.float32)


def _rd(x):
    r = pltpu.roll(x, 1, axis=0)
    idx = jax.lax.broadcasted_iota(jnp.int32, (x.shape[0], 1), 0)
    return jnp.where((idx % _MOD) == 0, 0.0, r)


def _ru(x):
    r = pltpu.roll(x, x.shape[0] - 1, axis=0)
    idx = jax.lax.broadcasted_iota(jnp.int32, (x.shape[0], 1), 0)
    return jnp.where((idx % _MOD) == (_MOD - 1), 0.0, r)


def _lrd(x):
    r = pltpu.roll(x, 1, axis=2)
    idx = jax.lax.broadcasted_iota(jnp.int32, (1, 1, x.shape[2]), 2)
    return jnp.where((idx % _MOD) == 0, 0.0, r)


def _lru(x):
    r = pltpu.roll(x, x.shape[2] - 1, axis=2)
    idx = jax.lax.broadcasted_iota(jnp.int32, (1, 1, x.shape[2]), 2)
    return jnp.where((idx % _MOD) == (_MOD - 1), 0.0, r)


def _cdot(x3, w):
    return jax.lax.dot_general(x3, w, (((1,), (1,)), ((), ())),
                               preferred_element_type=jnp.float32)


def _merge(t3):
    return t3.reshape(1024, t3.shape[2])


def _bn_fold(g, bt, m, v, b):
    s = g * jax.lax.rsqrt(v + 1e-5)
    return s[None, :], ((b - m) * s + bt)[None, :]


def _encode_all(x_ref, wpack, w2pack, bpack, out_ref):
    X0 = x_ref[0]
    X1 = x_ref[1]
    X2 = x_ref[2]
    X3 = x_ref[3]

    w2 = w2pack[...]
    wr1 = w2[0, 0:64, 0:22]
    wr2 = w2[1, 0:128, 0:64]
    wr3 = w2[2, 0:32, 0:128]
    wf = w2[3, 0:22, 0:32]
    caw1 = w2[4, 0:11, 0:22]
    caw2 = w2[5, 0:22, 0:11]
    wq = w2[6, 0:64, 0:32]
    wk = w2[7, 0:64, 0:32]
    wv = w2[8, 0:64, 0:32]
    wo = w2[9, 0:64, 0:64]
    mi = w2[10, 0:32, 0:16]

    bp = bpack[...]
    b1r = bp[0:1, 0:64]
    g1r = bp[1:2, 0:64]
    bt1r = bp[2:3, 0:64]
    m1r = bp[3:4, 0:64]
    v1r = bp[4:5, 0:64]
    br1 = bp[5:6, 0:64]
    bd1r = bp[6:7, 0:64]
    br2r = bp[7:8, 0:128]
    b2r = bp[8:9, 0:128]
    g2r = bp[9:10, 0:128]
    bt2r = bp[10:11, 0:128]
    m2r = bp[11:12, 0:128]
    v2r = bp[12:13, 0:128]
    bd2r = bp[13:14, 0:32]
    br3r = bp[14:15, 0:32]
    bfr = bp[15:16, 0:22]
    bqr = bp[16:17, 0:64]
    bkr = bp[17:18, 0:64]
    bvr = bp[18:19, 0:64]
    bor = bp[19:20, 0:64]

    s1 = g1r * jax.lax.rsqrt(v1r + 1e-5)
    b1e = (b1r - m1r) * s1 + bt1r
    s2 = g2r * jax.lax.rsqrt(v2r + 1e-5)
    b2e = (b2r - m2r) * s2 + bt2r

    wp = wpack[...]
    w10 = wp[0, 0:64, 0:22]
    w11 = wp[0, 64:128, 0:22]
    w12 = wp[0, 128:192, 0:22]
    h0 = _merge(_cdot(_lrd(X3), w10) + _cdot(X0, w11) + _cdot(X1, w12))
    h1 = _merge(_cdot(X0, w10) + _cdot(X1, w11) + _cdot(X2, w12))
    h2 = _merge(_cdot(X1, w10) + _cdot(X2, w11) + _cdot(X3, w12))
    h3 = _merge(_cdot(X2, w10) + _cdot(X3, w11) + _cdot(_lru(X0), w12))
    i0 = _merge(_cdot(X0, wr1)) + br1
    i1 = _merge(_cdot(X1, wr1)) + br1
    i2r = _merge(_cdot(X2, wr1)) + br1
    i3r = _merge(_cdot(X3, wr1)) + br1
    h0 = jnp.maximum(jnp.maximum(h0 * s1 + b1e, 0.0) + i0, 0.0)
    h1 = jnp.maximum(jnp.maximum(h1 * s1 + b1e, 0.0) + i1, 0.0)
    h2 = jnp.maximum(jnp.maximum(h2 * s1 + b1e, 0.0) + i2r, 0.0)
    h3 = jnp.maximum(jnp.maximum(h3 * s1 + b1e, 0.0) + i3r, 0.0)

    v0 = wp[1, 0:64, 0:64]
    v1t = wp[1, 64:128, 0:64]
    v2t = wp[1, 128:192, 0:64]
    ye = _mm_tt(_rd(h3), v0) + _mm_tt(h0, v1t) + _mm_tt(h1, v2t) + bd1r
    yo = _mm_tt(h1, v0) + _mm_tt(h2, v1t) + _mm_tt(h3, v2t) + bd1r

    i2e = _mm_tt((i0 + i1) * 0.5, wr2) + br2r
    i2o = _mm_tt((i2r + i3r) * 0.5, wr2) + br2r

    c0 = wp[2, 0:128, 0:64]
    c1 = wp[2, 128:256, 0:64]
    c2 = wp[2, 256:384, 0:64]
    he = _mm_tt(_rd(yo), c0) + _mm_tt(ye, c1) + _mm_tt(yo, c2)
    ho = _mm_tt(ye, c0) + _mm_tt(yo, c1) + _mm_tt(_ru(ye), c2)
    h4e = jnp.maximum(jnp.maximum(he * s2 + b2e, 0.0) + i2e, 0.0)
    h4o = jnp.maximum(jnp.maximum(ho * s2 + b2e, 0.0) + i2o, 0.0)

    e0 = wp[3, 0:32, 0:128]
    e1 = wp[3, 32:64, 0:128]
    e2 = wp[3, 64:96, 0:128]
    z = _mm_tt(_rd(h4o), e0) + _mm_tt(h4e, e1) + _mm_tt(h4o, e2) + bd2r
    i3 = _mm_tt((i2e + i2o) * 0.5, wr3) + br3r
    z2 = jnp.maximum(z + i3, 0.0)

    z3 = z2.reshape(64, 16, 32)
    hi = jax.lax.dot_general(z3, mi, (((1,), (1,)), ((), ())),
                             preferred_element_type=jnp.float32)
    hf = jax.lax.dot_general(hi, wf, (((1,), (1,)), ((), ())),
                             preferred_element_type=jnp.float32)
    hf = hf + bfr[None, :, :]

    avg = jnp.mean(hf, axis=1)
    mx = jnp.max(hf, axis=1)
    ga = _mm_tt(jnp.maximum(_mm_tt(avg, caw1), 0.0), caw2)
    gm = _mm_tt(jnp.maximum(_mm_tt(mx, caw1), 0.0), caw2)
    gate = jax.nn.sigmoid(ga + gm)
    o = jnp.sum(hf * gate[:, None, :], axis=2)

    o3 = o.reshape(8, 8, 32)
    q = jax.lax.dot_general(o3, wq, (((2,), (1,)), ((), ())),
                            preferred_element_type=jnp.float32) + bqr[None, :, :]
    k = jax.lax.dot_general(o3, wk, (((2,), (1,)), ((), ())),
                            preferred_element_type=jnp.float32) + bkr[None, :, :]
    v = jax.lax.dot_general(o3, wv, (((2,), (1,)), ((), ())),
                            preferred_element_type=jnp.float32) + bvr[None, :, :]
    sc = jax.lax.dot_general(q, k, (((2,), (2,)), ((0,), (0,))),
                             preferred_element_type=jnp.float32) * 0.125
    sc = sc - jnp.max(sc, axis=2, keepdims=True)
    es = jnp.exp(sc)
    p = es / jnp.sum(es, axis=2, keepdims=True)
    wvv = jax.lax.dot_general(p, v, (((2,), (1,)), ((0,), (0,))),
                              preferred_element_type=jnp.float32)
    pooled = jnp.mean(wvv, axis=1)
    out_ref[...] = _mm_tt(pooled, wo) + bor


def kernel(x, params):
    p = params
    f32 = jnp.float32

    def kstack(w):
        t = jnp.transpose(w, (2, 0, 1)).reshape(3 * w.shape[0], w.shape[1])
        return jnp.pad(t, ((0, 384 - t.shape[0]), (0, 128 - t.shape[1])))

    wpack = jnp.stack([kstack(p['w1']), kstack(p['wd1']),
                       kstack(p['w2']), kstack(p['wd2'])])

    def pad2(w):
        return jnp.pad(w, ((0, 128 - w.shape[0]), (0, 128 - w.shape[1])))

    w2pack = jnp.stack([pad2(p['wr1'][:, :, 0]), pad2(p['wr2'][:, :, 0]),
                        pad2(p['wr3'][:, :, 0]), pad2(p['wf'][:, :, 0]),
                        pad2(p['ca_w1']), pad2(p['ca_w2']),
                        pad2(p['wq']), pad2(p['wk']), pad2(p['wv']),
                        pad2(p['wo']), pad2(jnp.asarray(_MI))])

    def bpad(b):
        return jnp.pad(b, (0, 128 - b.shape[0]))

    bpack = jnp.stack([
        bpad(p['b1']), bpad(p['bn1_g']), bpad(p['bn1_b']), bpad(p['bn1_m']),
        bpad(p['bn1_v']), bpad(p['br1']), bpad(p['bd1']), bpad(p['br2']),
        bpad(p['b2']), bpad(p['bn2_g']), bpad(p['bn2_b']), bpad(p['bn2_m']),
        bpad(p['bn2_v']), bpad(p['bd2']), bpad(p['br3']), bpad(p['bf']),
        bpad(p['bq']), bpad(p['bk']), bpad(p['bv']), bpad(p['bo']),
        bpad(p['b1']), bpad(p['b1']), bpad(p['b1']), bpad(p['b1']),
    ])

    xph = jnp.transpose(x.reshape(8, 22, 128, 4), (3, 0, 1, 2))

    specs = [
        pl.BlockSpec((4, 8, 22, 128), lambda: (0, 0, 0, 0)),
        pl.BlockSpec((4, 384, 128), lambda: (0, 0, 0)),
        pl.BlockSpec((11, 128, 128), lambda: (0, 0, 0)),
        pl.BlockSpec((24, 128), lambda: (0, 0)),
    ]
    out = pl.pallas_call(
        _encode_all,
        in_specs=specs,
        out_specs=pl.BlockSpec((8, 64), lambda: (0, 0)),
        out_shape=jax.ShapeDtypeStruct((8, 64), f32),
    )(xph, wpack, w2pack, bpack)
    return out
